# Initial kernel scaffold; baseline (speedup 1.0000x reference)
import jax
import jax.numpy as jnp
from jax.experimental import pallas as pl


def kernel(inputs, roi_label, lut, cq):
    raise NotImplementedError("write your pallas kernel here")


# speedup vs baseline: 3.6044x; 3.6044x over previous
import functools

import jax
import jax.numpy as jnp
from jax.experimental import pallas as pl
from jax.experimental.pallas import tpu as pltpu

NUM_PIDS = 100000
NUM_CQ = 5000
NUM_FEAT = 128
BATCH = 1024
OIM_SCALAR = 30.0
IGNORE_INDEX = 5554

BLOCK_N = 2048
NUM_BLOCKS = (NUM_PIDS + BLOCK_N - 1) // BLOCK_N

_NEG = -1e30


def _oim_kernel(x_ref, lab_ref, lut_ref, cq_ref, out_ref, m_ref, s_ref, p_ref):
    j = pl.program_id(0)

    @pl.when(j == 0)
    def _init():
        m_ref[...] = jnp.full((BATCH, 1), _NEG, jnp.float32)
        s_ref[...] = jnp.zeros((BATCH, 1), jnp.float32)
        p_ref[...] = jnp.zeros((BATCH, 1), jnp.float32)

    x = x_ref[...]
    w = lut_ref[...]
    logits = jax.lax.dot_general(
        x, w, (((1,), (1,)), ((), ())),
        preferred_element_type=jnp.float32) * OIM_SCALAR

    col = j * BLOCK_N + jax.lax.broadcasted_iota(jnp.int32, (BATCH, BLOCK_N), 1)
    logits = jnp.where(col < NUM_PIDS, logits, _NEG)

    lab = lab_ref[...]
    label_all = lab - 1
    keep = label_all >= 0
    label_idx = jnp.where(keep, label_all, 0)

    m_old = m_ref[...]
    bm = jnp.max(logits, axis=1, keepdims=True)
    m_new = jnp.maximum(m_old, bm)
    alpha = jnp.exp(m_old - m_new)
    e = jnp.exp(logits - m_new)
    s_ref[...] = s_ref[...] * alpha + jnp.sum(e, axis=1, keepdims=True)
    m_ref[...] = m_new
    p_ref[...] = p_ref[...] + jnp.sum(
        jnp.where(col == label_idx, logits, 0.0), axis=1, keepdims=True)

    @pl.when(j == NUM_BLOCKS - 1)
    def _tail():
        cq = cq_ref[...]
        logits2 = jax.lax.dot_general(
            x, cq, (((1,), (1,)), ((), ())),
            preferred_element_type=jnp.float32) * OIM_SCALAR
        m_old2 = m_ref[...]
        bm2 = jnp.max(logits2, axis=1, keepdims=True)
        m2 = jnp.maximum(m_old2, bm2)
        e2 = jnp.exp(logits2 - m2)
        s = s_ref[...] * jnp.exp(m_old2 - m2) + jnp.sum(e2, axis=1, keepdims=True)
        ce = m2 + jnp.log(s) - p_ref[...]
        valid = jnp.logical_and(keep, label_all != IGNORE_INDEX)
        vf = valid.astype(jnp.float32)
        denom = jnp.maximum(jnp.sum(vf), 1.0)
        out_ref[...] = jnp.sum(ce * vf, keepdims=True).reshape(1, 1) / denom


@functools.partial(jax.jit, static_argnames=())
def _oim_loss(inputs, roi_label, lut, cq):
    lab2d = roi_label.reshape(BATCH, 1)
    out = pl.pallas_call(
        _oim_kernel,
        grid=(NUM_BLOCKS,),
        in_specs=[
            pl.BlockSpec((BATCH, NUM_FEAT), lambda j: (0, 0)),
            pl.BlockSpec((BATCH, 1), lambda j: (0, 0)),
            pl.BlockSpec((BLOCK_N, NUM_FEAT), lambda j: (j, 0)),
            pl.BlockSpec((NUM_CQ, NUM_FEAT), lambda j: (0, 0)),
        ],
        out_specs=pl.BlockSpec((1, 1), lambda j: (0, 0)),
        out_shape=jax.ShapeDtypeStruct((1, 1), jnp.float32),
        scratch_shapes=[
            pltpu.VMEM((BATCH, 1), jnp.float32),
            pltpu.VMEM((BATCH, 1), jnp.float32),
            pltpu.VMEM((BATCH, 1), jnp.float32),
        ],
    )(inputs.astype(jnp.float32), lab2d, lut.astype(jnp.float32),
      cq.astype(jnp.float32))
    return out.reshape(())


def kernel(inputs, roi_label, lut, cq):
    return _oim_loss(inputs, roi_label, lut, cq)


# speedup vs baseline: 4.0116x; 1.1130x over previous
import functools

import jax
import jax.numpy as jnp
from jax.experimental import pallas as pl
from jax.experimental.pallas import tpu as pltpu

NUM_PIDS = 100000
NUM_CQ = 5000
NUM_FEAT = 128
BATCH = 1024
OIM_SCALAR = 30.0
IGNORE_INDEX = 5554

BLOCK_N = 2000
NUM_BLOCKS = NUM_PIDS // BLOCK_N

_NEG = -1e30


def _oim_kernel(x_ref, lab_ref, lut_ref, cq_ref, out_ref, m_ref, s_ref, p_ref):
    j = pl.program_id(0)

    @pl.when(j == 0)
    def _init():
        m_ref[...] = jnp.full((BATCH, 1), _NEG, jnp.float32)
        s_ref[...] = jnp.zeros((BATCH, 1), jnp.float32)
        p_ref[...] = jnp.zeros((BATCH, 1), jnp.float32)

    x = x_ref[...] * OIM_SCALAR
    w = lut_ref[...]
    logits = jax.lax.dot_general(
        x, w, (((1,), (1,)), ((), ())),
        preferred_element_type=jnp.float32)

    lab = lab_ref[...]
    label_all = lab - 1
    keep = label_all >= 0
    label_idx = jnp.where(keep, label_all, 0)
    rel = label_idx - j * BLOCK_N
    col = jax.lax.broadcasted_iota(jnp.int32, (BATCH, BLOCK_N), 1)

    m_old = m_ref[...]
    bm = jnp.max(logits, axis=1, keepdims=True)
    m_new = jnp.maximum(m_old, bm)
    alpha = jnp.exp(m_old - m_new)
    e = jnp.exp(logits - m_new)
    s_ref[...] = s_ref[...] * alpha + jnp.sum(e, axis=1, keepdims=True)
    m_ref[...] = m_new
    p_ref[...] = p_ref[...] + jnp.sum(
        jnp.where(col == rel, logits, 0.0), axis=1, keepdims=True)

    @pl.when(j == NUM_BLOCKS - 1)
    def _tail():
        cq = cq_ref[...]
        logits2 = jax.lax.dot_general(
            x, cq, (((1,), (1,)), ((), ())),
            preferred_element_type=jnp.float32)
        m_old2 = m_ref[...]
        bm2 = jnp.max(logits2, axis=1, keepdims=True)
        m2 = jnp.maximum(m_old2, bm2)
        e2 = jnp.exp(logits2 - m2)
        s = s_ref[...] * jnp.exp(m_old2 - m2) + jnp.sum(e2, axis=1, keepdims=True)
        ce = m2 + jnp.log(s) - p_ref[...]
        valid = jnp.logical_and(keep, label_all != IGNORE_INDEX)
        vf = valid.astype(jnp.float32)
        denom = jnp.maximum(jnp.sum(vf), 1.0)
        out_ref[...] = jnp.sum(ce * vf, keepdims=True).reshape(1, 1) / denom


@functools.partial(jax.jit, static_argnames=())
def _oim_loss(inputs, roi_label, lut, cq):
    lab2d = roi_label.reshape(BATCH, 1)
    out = pl.pallas_call(
        _oim_kernel,
        grid=(NUM_BLOCKS,),
        in_specs=[
            pl.BlockSpec((BATCH, NUM_FEAT), lambda j: (0, 0)),
            pl.BlockSpec((BATCH, 1), lambda j: (0, 0)),
            pl.BlockSpec((BLOCK_N, NUM_FEAT), lambda j: (j, 0)),
            pl.BlockSpec((NUM_CQ, NUM_FEAT), lambda j: (0, 0)),
        ],
        out_specs=pl.BlockSpec((1, 1), lambda j: (0, 0)),
        out_shape=jax.ShapeDtypeStruct((1, 1), jnp.float32),
        scratch_shapes=[
            pltpu.VMEM((BATCH, 1), jnp.float32),
            pltpu.VMEM((BATCH, 1), jnp.float32),
            pltpu.VMEM((BATCH, 1), jnp.float32),
        ],
    )(inputs.astype(jnp.float32), lab2d, lut.astype(jnp.float32),
      cq.astype(jnp.float32))
    return out.reshape(())


def kernel(inputs, roi_label, lut, cq):
    return _oim_loss(inputs, roi_label, lut, cq)


# speedup vs baseline: 4.0398x; 1.0070x over previous
import functools

import jax
import jax.numpy as jnp
from jax import lax
from jax.experimental import pallas as pl
from jax.experimental.pallas import tpu as pltpu
from jax.experimental.pallas import tpu_sc as plsc

NUM_PIDS = 100000
NUM_CQ = 5000
NUM_FEAT = 128
BATCH = 1024
OIM_SCALAR = 30.0
IGNORE_INDEX = 5554

BLOCK_N = 2000
NUM_BLOCKS = NUM_PIDS // BLOCK_N

_NEG = -1e30


def _lse_kernel(x_ref, lut_ref, cq_ref, ls_ref, m_ref, s_ref):
    j = pl.program_id(0)

    @pl.when(j == 0)
    def _init():
        m_ref[...] = jnp.full((BATCH, 1), _NEG, jnp.float32)
        s_ref[...] = jnp.zeros((BATCH, 1), jnp.float32)

    x = x_ref[...] * OIM_SCALAR
    w = lut_ref[...]
    logits = jax.lax.dot_general(
        x, w, (((1,), (1,)), ((), ())),
        preferred_element_type=jnp.float32)

    m_old = m_ref[...]
    bm = jnp.max(logits, axis=1, keepdims=True)
    m_new = jnp.maximum(m_old, bm)
    e = jnp.exp(logits - m_new)
    s_ref[...] = s_ref[...] * jnp.exp(m_old - m_new) + jnp.sum(
        e, axis=1, keepdims=True)
    m_ref[...] = m_new

    @pl.when(j == NUM_BLOCKS - 1)
    def _tail():
        cq = cq_ref[...]
        logits2 = jax.lax.dot_general(
            x, cq, (((1,), (1,)), ((), ())),
            preferred_element_type=jnp.float32)
        m_old2 = m_ref[...]
        bm2 = jnp.max(logits2, axis=1, keepdims=True)
        m2 = jnp.maximum(m_old2, bm2)
        e2 = jnp.exp(logits2 - m2)
        s = s_ref[...] * jnp.exp(m_old2 - m2) + jnp.sum(e2, axis=1, keepdims=True)
        ls_ref[...] = m2 + jnp.log(s)


@functools.cache
def _sc_gather_fn():
    info = plsc.get_sparse_core_info()
    nc, ns = info.num_cores, info.num_subcores
    bpw = BATCH // (nc * ns)
    mesh = plsc.VectorSubcoreMesh(core_axis_name="c", subcore_axis_name="s")

    @functools.partial(
        pl.kernel,
        mesh=mesh,
        out_type=jax.ShapeDtypeStruct((BATCH, NUM_FEAT), jnp.float32),
        scratch_types=[
            pltpu.VMEM((bpw,), jnp.int32),
            pltpu.VMEM((bpw, NUM_FEAT), jnp.float32),
            pltpu.SemaphoreType.DMA,
        ],
    )
    def _sc_gather(lab_hbm, lut_hbm, out_hbm, idx_v, rows_v, sem):
        wid = lax.axis_index("s") * nc + lax.axis_index("c")
        base = wid * bpw
        pltpu.sync_copy(lab_hbm.at[pl.ds(base, bpw)], idx_v)
        for k in range(bpw // 16):
            v = idx_v[pl.ds(k * 16, 16)]
            idx_v[pl.ds(k * 16, 16)] = jnp.maximum(v - 1, 0)
        pltpu.async_copy(lut_hbm.at[idx_v], rows_v, sem).wait()
        pltpu.sync_copy(rows_v, out_hbm.at[pl.ds(base, bpw)])

    return _sc_gather


def _loss_kernel(ls_ref, g_ref, x_ref, lab_ref, out_ref):
    xs = x_ref[...] * OIM_SCALAR
    picked = jnp.sum(xs * g_ref[...], axis=1, keepdims=True)
    lab = lab_ref[...]
    label_all = lab - 1
    valid = jnp.logical_and(label_all >= 0, label_all != IGNORE_INDEX)
    vf = valid.astype(jnp.float32)
    denom = jnp.maximum(jnp.sum(vf), 1.0)
    ce = ls_ref[...] - picked
    out_ref[...] = jnp.sum(ce * vf, keepdims=True).reshape(1, 1) / denom


@jax.jit
def _oim_loss(inputs, roi_label, lut, cq):
    x = inputs.astype(jnp.float32)
    lut = lut.astype(jnp.float32)
    cq = cq.astype(jnp.float32)
    lab2d = roi_label.reshape(BATCH, 1)

    ls = pl.pallas_call(
        _lse_kernel,
        grid=(NUM_BLOCKS,),
        in_specs=[
            pl.BlockSpec((BATCH, NUM_FEAT), lambda j: (0, 0)),
            pl.BlockSpec((BLOCK_N, NUM_FEAT), lambda j: (j, 0)),
            pl.BlockSpec((NUM_CQ, NUM_FEAT), lambda j: (0, 0)),
        ],
        out_specs=pl.BlockSpec((BATCH, 1), lambda j: (0, 0)),
        out_shape=jax.ShapeDtypeStruct((BATCH, 1), jnp.float32),
        scratch_shapes=[
            pltpu.VMEM((BATCH, 1), jnp.float32),
            pltpu.VMEM((BATCH, 1), jnp.float32),
        ],
    )(x, lut, cq)

    g = _sc_gather_fn()(roi_label, lut)

    out = pl.pallas_call(
        _loss_kernel,
        in_specs=[
            pl.BlockSpec((BATCH, 1), lambda: (0, 0)),
            pl.BlockSpec((BATCH, NUM_FEAT), lambda: (0, 0)),
            pl.BlockSpec((BATCH, NUM_FEAT), lambda: (0, 0)),
            pl.BlockSpec((BATCH, 1), lambda: (0, 0)),
        ],
        out_specs=pl.BlockSpec((1, 1), lambda: (0, 0)),
        out_shape=jax.ShapeDtypeStruct((1, 1), jnp.float32),
    )(ls, g, x, lab2d)
    return out.reshape(())


def kernel(inputs, roi_label, lut, cq):
    return _oim_loss(inputs, roi_label, lut, cq)


# speedup vs baseline: 4.9949x; 1.2364x over previous
import functools

import jax
import jax.numpy as jnp
from jax import lax
from jax.experimental import pallas as pl
from jax.experimental.pallas import tpu as pltpu
from jax.experimental.pallas import tpu_sc as plsc

NUM_PIDS = 100000
NUM_CQ = 5000
NUM_FEAT = 128
BATCH = 1024
OIM_SCALAR = 30.0
IGNORE_INDEX = 5554

BLOCK_N = 4000
NUM_BLOCKS = NUM_PIDS // BLOCK_N

_NEG = -1e30
_LOG2E = 1.4426950408889634
_LN2 = 0.6931471805599453


def _lse_kernel(x_ref, lut_ref, cq_ref, ls_ref, m_ref, s_ref):
    j = pl.program_id(0)

    @pl.when(j == 0)
    def _init():
        m_ref[...] = jnp.full((BATCH, 1), _NEG, jnp.float32)
        s_ref[...] = jnp.zeros((BATCH, 1), jnp.float32)

    x = x_ref[...] * (OIM_SCALAR * _LOG2E)
    w = lut_ref[...]
    logits = jax.lax.dot_general(
        x, w, (((1,), (1,)), ((), ())),
        preferred_element_type=jnp.float32)

    m_old = m_ref[...]
    bm = jnp.max(logits, axis=1, keepdims=True)
    m_new = jnp.maximum(m_old, bm)
    e = jnp.exp2(logits - m_new)
    s_ref[...] = s_ref[...] * jnp.exp2(m_old - m_new) + jnp.sum(
        e, axis=1, keepdims=True)
    m_ref[...] = m_new

    @pl.when(j == NUM_BLOCKS - 1)
    def _tail():
        cq = cq_ref[...]
        logits2 = jax.lax.dot_general(
            x, cq, (((1,), (1,)), ((), ())),
            preferred_element_type=jnp.float32)
        m_old2 = m_ref[...]
        bm2 = jnp.max(logits2, axis=1, keepdims=True)
        m2 = jnp.maximum(m_old2, bm2)
        e2 = jnp.exp2(logits2 - m2)
        s = s_ref[...] * jnp.exp2(m_old2 - m2) + jnp.sum(e2, axis=1, keepdims=True)
        ls_ref[...] = (m2 + jnp.log2(s)) * _LN2


@functools.cache
def _sc_gather_fn():
    info = plsc.get_sparse_core_info()
    nc, ns = info.num_cores, info.num_subcores
    bpw = BATCH // (nc * ns)
    mesh = plsc.VectorSubcoreMesh(core_axis_name="c", subcore_axis_name="s")

    @functools.partial(
        pl.kernel,
        mesh=mesh,
        out_type=jax.ShapeDtypeStruct((BATCH, NUM_FEAT), jnp.float32),
        scratch_types=[
            pltpu.VMEM((bpw,), jnp.int32),
            pltpu.VMEM((bpw, NUM_FEAT), jnp.float32),
            pltpu.SemaphoreType.DMA,
        ],
    )
    def _sc_gather(lab_hbm, lut_hbm, out_hbm, idx_v, rows_v, sem):
        wid = lax.axis_index("s") * nc + lax.axis_index("c")
        base = wid * bpw
        pltpu.sync_copy(lab_hbm.at[pl.ds(base, bpw)], idx_v)
        for k in range(bpw // 16):
            v = idx_v[pl.ds(k * 16, 16)]
            idx_v[pl.ds(k * 16, 16)] = jnp.maximum(v - 1, 0)
        pltpu.async_copy(lut_hbm.at[idx_v], rows_v, sem).wait()
        pltpu.sync_copy(rows_v, out_hbm.at[pl.ds(base, bpw)])

    return _sc_gather


def _loss_kernel(ls_ref, g_ref, x_ref, lab_ref, out_ref):
    xs = x_ref[...] * OIM_SCALAR
    picked = jnp.sum(xs * g_ref[...], axis=1, keepdims=True)
    lab = lab_ref[...]
    label_all = lab - 1
    valid = jnp.logical_and(label_all >= 0, label_all != IGNORE_INDEX)
    vf = valid.astype(jnp.float32)
    denom = jnp.maximum(jnp.sum(vf), 1.0)
    ce = ls_ref[...] - picked
    out_ref[...] = jnp.sum(ce * vf, keepdims=True).reshape(1, 1) / denom


@jax.jit
def _oim_loss(inputs, roi_label, lut, cq):
    x = inputs.astype(jnp.float32)
    lut = lut.astype(jnp.float32)
    cq = cq.astype(jnp.float32)
    lab2d = roi_label.reshape(BATCH, 1)

    ls = pl.pallas_call(
        _lse_kernel,
        grid=(NUM_BLOCKS,),
        in_specs=[
            pl.BlockSpec((BATCH, NUM_FEAT), lambda j: (0, 0)),
            pl.BlockSpec((BLOCK_N, NUM_FEAT), lambda j: (j, 0)),
            pl.BlockSpec((NUM_CQ, NUM_FEAT), lambda j: (0, 0)),
        ],
        out_specs=pl.BlockSpec((BATCH, 1), lambda j: (0, 0)),
        out_shape=jax.ShapeDtypeStruct((BATCH, 1), jnp.float32),
        scratch_shapes=[
            pltpu.VMEM((BATCH, 1), jnp.float32),
            pltpu.VMEM((BATCH, 1), jnp.float32),
        ],
    )(x, lut, cq)

    g = _sc_gather_fn()(roi_label, lut)

    out = pl.pallas_call(
        _loss_kernel,
        in_specs=[
            pl.BlockSpec((BATCH, 1), lambda: (0, 0)),
            pl.BlockSpec((BATCH, NUM_FEAT), lambda: (0, 0)),
            pl.BlockSpec((BATCH, NUM_FEAT), lambda: (0, 0)),
            pl.BlockSpec((BATCH, 1), lambda: (0, 0)),
        ],
        out_specs=pl.BlockSpec((1, 1), lambda: (0, 0)),
        out_shape=jax.ShapeDtypeStruct((1, 1), jnp.float32),
    )(ls, g, x, lab2d)
    return out.reshape(())


def kernel(inputs, roi_label, lut, cq):
    return _oim_loss(inputs, roi_label, lut, cq)
